# baseline (speedup 1.0000x reference)
import functools

import jax
import jax.numpy as jnp
from jax import lax
from jax.experimental import pallas as pl
from jax.experimental.pallas import tpu as pltpu
from jax.experimental.pallas import tpu_sc as plsc

VOCAB = 100000
D_MODEL = 128
BATCH = 4096
SEQ = 50
NUM_CORES = 2
NUM_SUBCORES = 16
NW = NUM_CORES * NUM_SUBCORES
NSPLIT = 2
HALF = BATCH // NSPLIT
B_PER_W = HALF // NW
NBUF = 8
GDEPTH = 4

_mesh = plsc.VectorSubcoreMesh(core_axis_name="c", subcore_axis_name="s")


def _make_half(base0):
    @functools.partial(
        pl.kernel,
        mesh=_mesh,
        out_type=jax.ShapeDtypeStruct((HALF, SEQ, D_MODEL), jnp.float32),
        scratch_types=(
            [pltpu.VMEM((B_PER_W, SEQ), jnp.int32)]
            + [pltpu.VMEM((SEQ, D_MODEL), jnp.float32)] * NBUF
            + [pltpu.SemaphoreType.DMA] * (2 * NBUF)
        ),
    )
    def _embed(x_hbm, table_hbm, out_hbm, idx_v, *bufs_and_sems):
        rows = bufs_and_sems[:NBUF]
        gsem = bufs_and_sems[NBUF:2 * NBUF]
        wsem = bufs_and_sems[2 * NBUF:]

        wid = lax.axis_index("s") * NUM_CORES + lax.axis_index("c")
        base = wid * B_PER_W

        pltpu.sync_copy(x_hbm.at[pl.ds(base0 + base, B_PER_W)], idx_v)

        for i in range(GDEPTH):
            pltpu.async_copy(table_hbm.at[idx_v.at[i]], rows[i], gsem[i])

        def group(g, carry):
            for s in range(NBUF):
                j = g * NBUF + s

                pltpu.make_async_copy(
                    table_hbm.at[idx_v.at[j]], rows[s], gsem[s]
                ).wait()
                pltpu.async_copy(rows[s], out_hbm.at[base + j], wsem[s])

                kb = (s + GDEPTH) % NBUF

                @pl.when(j + GDEPTH < B_PER_W)
                def _():
                    @pl.when(j >= NBUF - GDEPTH)
                    def _():
                        pltpu.make_async_copy(
                            rows[kb], out_hbm.at[0], wsem[kb]
                        ).wait()
                    pltpu.async_copy(
                        table_hbm.at[idx_v.at[j + GDEPTH]], rows[kb], gsem[kb]
                    )
            return carry

        lax.fori_loop(0, B_PER_W // NBUF, group, 0)

        for s in range(NBUF):
            pltpu.make_async_copy(rows[s], out_hbm.at[0], wsem[s]).wait()

    return _embed


_embed_halves = [_make_half(h * HALF) for h in range(NSPLIT)]


def kernel(x, table):
    parts = [f(x, table) for f in _embed_halves]
    return jnp.concatenate(parts, axis=0)


# speedup vs baseline: 1.6166x; 1.6166x over previous
import functools

import jax
import jax.numpy as jnp
from jax import lax
from jax.experimental import pallas as pl
from jax.experimental.pallas import tpu as pltpu
from jax.experimental.pallas import tpu_sc as plsc

VOCAB = 100000
D_MODEL = 128
BATCH = 4096
SEQ = 50
NUM_CORES = 2
NUM_SUBCORES = 16
NW = NUM_CORES * NUM_SUBCORES
B_PER_W = BATCH // NW
NBUF = 8
GDEPTH = 4

_mesh = plsc.VectorSubcoreMesh(core_axis_name="c", subcore_axis_name="s")


@functools.partial(
    pl.kernel,
    mesh=_mesh,
    out_type=jax.ShapeDtypeStruct((BATCH, SEQ, D_MODEL), jnp.float32),
    scratch_types=(
        [pltpu.VMEM((B_PER_W, SEQ), jnp.int32)]
        + [pltpu.VMEM((SEQ, D_MODEL), jnp.float32)] * NBUF
        + [pltpu.SemaphoreType.DMA] * (2 * NBUF)
    ),
    compiler_params=pltpu.CompilerParams(use_tc_tiling_on_sc=True),
)
def _embed(x_hbm, table_hbm, out_hbm, idx_v, *bufs_and_sems):
    rows = bufs_and_sems[:NBUF]
    gsem = bufs_and_sems[NBUF:2 * NBUF]
    wsem = bufs_and_sems[2 * NBUF:]

    wid = lax.axis_index("s") * NUM_CORES + lax.axis_index("c")
    base = wid * B_PER_W

    pltpu.sync_copy(x_hbm.at[pl.ds(base, B_PER_W)], idx_v)

    for i in range(GDEPTH):
        pltpu.async_copy(table_hbm.at[idx_v.at[i]], rows[i], gsem[i])

    def group(g, carry):
        for s in range(NBUF):
            j = g * NBUF + s

            pltpu.make_async_copy(
                table_hbm.at[idx_v.at[j]], rows[s], gsem[s]
            ).wait()
            pltpu.async_copy(rows[s], out_hbm.at[base + j], wsem[s])

            kb = (s + GDEPTH) % NBUF

            @pl.when(j + GDEPTH < B_PER_W)
            def _():
                @pl.when(j >= NBUF - GDEPTH)
                def _():
                    pltpu.make_async_copy(
                        rows[kb], out_hbm.at[0], wsem[kb]
                    ).wait()
                pltpu.async_copy(
                    table_hbm.at[idx_v.at[j + GDEPTH]], rows[kb], gsem[kb]
                )
        return carry

    lax.fori_loop(0, B_PER_W // NBUF, group, 0)

    for s in range(NBUF):
        pltpu.make_async_copy(rows[s], out_hbm.at[0], wsem[s]).wait()


def kernel(x, table):
    return _embed(x, table)


# speedup vs baseline: 2.9018x; 1.7950x over previous
import functools

import jax
import jax.numpy as jnp
from jax import lax
from jax.experimental import pallas as pl
from jax.experimental.pallas import tpu as pltpu
from jax.experimental.pallas import tpu_sc as plsc

VOCAB = 100000
D_MODEL = 128
BATCH = 4096
SEQ = 50
NUM_CORES = 2
NUM_SUBCORES = 16
NW = NUM_CORES * NUM_SUBCORES
B_PER_W = BATCH // NW
NBUF = 5
GDEPTH = 3

_mesh = plsc.VectorSubcoreMesh(core_axis_name="c", subcore_axis_name="s")


@functools.partial(
    pl.kernel,
    mesh=_mesh,
    out_type=jax.ShapeDtypeStruct((SEQ, BATCH, D_MODEL), jnp.float32),
    scratch_types=(
        [pltpu.VMEM((SEQ, B_PER_W), jnp.int32)]
        + [pltpu.VMEM((B_PER_W, D_MODEL), jnp.float32)] * NBUF
        + [pltpu.SemaphoreType.DMA] * (2 * NBUF)
    ),
)
def _embed(x_hbm, table_hbm, out_hbm, idx_v, *bufs_and_sems):
    rows = bufs_and_sems[:NBUF]
    gsem = bufs_and_sems[NBUF:2 * NBUF]
    wsem = bufs_and_sems[2 * NBUF:]

    wid = lax.axis_index("s") * NUM_CORES + lax.axis_index("c")
    base = wid * B_PER_W

    pltpu.sync_copy(x_hbm.at[:, pl.ds(base, B_PER_W)], idx_v)

    for i in range(GDEPTH):
        pltpu.async_copy(table_hbm.at[idx_v.at[i]], rows[i], gsem[i])

    def group(g, carry):
        for s in range(NBUF):
            j = g * NBUF + s

            pltpu.make_async_copy(
                table_hbm.at[idx_v.at[j]], rows[s], gsem[s]
            ).wait()
            pltpu.async_copy(
                rows[s], out_hbm.at[j].at[pl.ds(base, B_PER_W)], wsem[s]
            )

            kb = (s + GDEPTH) % NBUF

            @pl.when(j + GDEPTH < SEQ)
            def _():
                @pl.when(j >= NBUF - GDEPTH)
                def _():
                    pltpu.make_async_copy(
                        rows[kb], out_hbm.at[0].at[pl.ds(0, B_PER_W)],
                        wsem[kb]
                    ).wait()
                pltpu.async_copy(
                    table_hbm.at[idx_v.at[j + GDEPTH]], rows[kb], gsem[kb]
                )
        return carry

    lax.fori_loop(0, SEQ // NBUF, group, 0)

    for s in range(NBUF):
        pltpu.make_async_copy(
            rows[s], out_hbm.at[0].at[pl.ds(0, B_PER_W)], wsem[s]
        ).wait()


def kernel(x, table):
    out = _embed(x.T, table)
    return out.transpose(1, 0, 2)
